# baseline (speedup 1.0000x reference)
import functools

import jax
import jax.numpy as jnp
from jax import lax
from jax.experimental import pallas as pl
from jax.experimental.pallas import tpu as pltpu
from jax.experimental.pallas import tpu_sc as plsc

_MAX = 32
_B, _C, _H, _W = 4, 96, 224, 224
_V = 2 * _MAX + 1
_L = 16
_NCH = _C // _L
_HG = 112


def _sc_body(emb_hbm, tab_hbm, emb_v, tab_v):
    cid = lax.axis_index("c")
    sid = lax.axis_index("s")
    wid = sid * 2 + cid

    @pl.when(wid == 0)
    def _():
        pltpu.sync_copy(emb_hbm, emb_v)
        pltpu.sync_copy(emb_hbm.at[pl.ds(0, 64)], tab_v.at[pl.ds(80, 64)])
        e0 = [emb_v[0, pl.ds(c * _L, _L)] for c in range(_NCH)]
        e64 = [emb_v[_V - 1, pl.ds(c * _L, _L)] for c in range(_NCH)]

        def lo_fill(x, carry):
            for c in range(_NCH):
                tab_v[x, pl.ds(c * _L, _L)] = e0[c]
            return carry

        def hi_fill(x, carry):
            for c in range(_NCH):
                tab_v[x, pl.ds(c * _L, _L)] = e64[c]
            return carry

        lax.fori_loop(0, 80, lo_fill, 0)
        lax.fori_loop(144, _W, hi_fill, 0)
        pltpu.sync_copy(tab_v, tab_hbm)


_table = functools.partial(
    pl.kernel,
    mesh=plsc.VectorSubcoreMesh(core_axis_name="c", subcore_axis_name="s"),
    out_type=jax.ShapeDtypeStruct((_W, _C), jnp.float32),
    scratch_types=[
        pltpu.VMEM((_V, _C), jnp.float32),
        pltpu.VMEM((_W, _C), jnp.float32),
    ],
)(_sc_body)


def _tc_body(wtile_ref, hcols_ref, out_ref, acc_ref):
    b = pl.program_id(1)

    @pl.when(b == 0)
    def _():
        hc = hcols_ref[0]
        hrep = jnp.concatenate(
            [jnp.broadcast_to(hc[:, i:i + 1], (_C, _W)) for i in range(_HG)],
            axis=1)
        acc_ref[...] = wtile_ref[...] + hrep

    out_ref[0] = acc_ref[...]


_fanout = pl.pallas_call(
    _tc_body,
    grid=(_H // _HG, _B),
    in_specs=[
        pl.BlockSpec((_C, _HG * _W), lambda k, b: (0, 0)),
        pl.BlockSpec((1, _C, _HG), lambda k, b: (k, 0, 0)),
    ],
    out_specs=pl.BlockSpec((1, _C, _HG * _W), lambda k, b: (b, 0, k)),
    out_shape=jax.ShapeDtypeStruct((_B, _C, _H * _W), jnp.float32),
    scratch_shapes=[pltpu.VMEM((_C, _HG * _W), jnp.float32)],
    compiler_params=pltpu.CompilerParams(
        dimension_semantics=("arbitrary", "arbitrary")),
)


def kernel(feature_map, embedding):
    del feature_map
    tabt = _table(embedding).T
    hcols = tabt.reshape(_C, _H // _HG, _HG).swapaxes(0, 1)
    wtile = jnp.tile(tabt, (1, _HG))
    return jnp.swapaxes(_fanout(wtile, hcols), 1, 2)


# speedup vs baseline: 1.2237x; 1.2237x over previous
import functools

import jax
import jax.numpy as jnp
from jax import lax
from jax.experimental import pallas as pl
from jax.experimental.pallas import tpu as pltpu
from jax.experimental.pallas import tpu_sc as plsc

_MAX = 32
_B, _C, _H, _W = 4, 96, 224, 224
_V = 2 * _MAX + 1
_L = 16
_NCH = _C // _L
_HG = 56


def _sc_body(emb_hbm, tab_hbm, emb_v, tab_v):
    cid = lax.axis_index("c")
    sid = lax.axis_index("s")
    wid = sid * 2 + cid

    @pl.when(wid == 0)
    def _():
        pltpu.sync_copy(emb_hbm, emb_v)
        pltpu.sync_copy(emb_hbm.at[pl.ds(0, 64)], tab_v.at[pl.ds(80, 64)])
        e0 = [emb_v[0, pl.ds(c * _L, _L)] for c in range(_NCH)]
        e64 = [emb_v[_V - 1, pl.ds(c * _L, _L)] for c in range(_NCH)]

        def lo_fill(x, carry):
            for c in range(_NCH):
                tab_v[x, pl.ds(c * _L, _L)] = e0[c]
            return carry

        def hi_fill(x, carry):
            for c in range(_NCH):
                tab_v[x, pl.ds(c * _L, _L)] = e64[c]
            return carry

        lax.fori_loop(0, 80, lo_fill, 0)
        lax.fori_loop(144, _W, hi_fill, 0)
        pltpu.sync_copy(tab_v, tab_hbm)


_table = functools.partial(
    pl.kernel,
    mesh=plsc.VectorSubcoreMesh(core_axis_name="c", subcore_axis_name="s"),
    out_type=jax.ShapeDtypeStruct((_W, _C), jnp.float32),
    scratch_types=[
        pltpu.VMEM((_V, _C), jnp.float32),
        pltpu.VMEM((_W, _C), jnp.float32),
    ],
)(_sc_body)


def _tc_body(wtile_ref, hcols_ref, out_ref, acc_ref):
    b = pl.program_id(1)

    @pl.when(b == 0)
    def _():
        hc = hcols_ref[0]
        hrep = jnp.concatenate(
            [jnp.broadcast_to(hc[:, i:i + 1], (_C, _W)) for i in range(_HG)],
            axis=1)
        acc_ref[...] = wtile_ref[...] + hrep

    out_ref[0] = acc_ref[...]


_fanout = pl.pallas_call(
    _tc_body,
    grid=(_H // _HG, _B),
    in_specs=[
        pl.BlockSpec((_C, _HG * _W), lambda k, b: (0, 0)),
        pl.BlockSpec((1, _C, _HG), lambda k, b: (k, 0, 0)),
    ],
    out_specs=pl.BlockSpec((1, _C, _HG * _W), lambda k, b: (b, 0, k)),
    out_shape=jax.ShapeDtypeStruct((_B, _C, _H * _W), jnp.float32),
    scratch_shapes=[pltpu.VMEM((_C, _HG * _W), jnp.float32)],
    compiler_params=pltpu.CompilerParams(
        dimension_semantics=("arbitrary", "arbitrary")),
)


def kernel(feature_map, embedding):
    del feature_map
    tabt = _table(embedding).T
    hcols = tabt.reshape(_C, _H // _HG, _HG).swapaxes(0, 1)
    wtile = jnp.tile(tabt, (1, _HG))
    return jnp.swapaxes(_fanout(wtile, hcols), 1, 2)


# speedup vs baseline: 1.2529x; 1.0238x over previous
import functools

import jax
import jax.numpy as jnp
from jax import lax
from jax.experimental import pallas as pl
from jax.experimental.pallas import tpu as pltpu
from jax.experimental.pallas import tpu_sc as plsc

_MAX = 32
_B, _C, _H, _W = 4, 96, 224, 224
_V = 2 * _MAX + 1
_L = 16
_NCH = _C // _L
_HG = 56


def _sc_body(emb_hbm, tab_hbm, emb_v, tab_v):
    cid = lax.axis_index("c")
    sid = lax.axis_index("s")
    wid = sid * 2 + cid

    @pl.when(wid == 0)
    def _():
        pltpu.sync_copy(emb_hbm, emb_v)
        pltpu.sync_copy(emb_hbm.at[pl.ds(0, 64)], tab_v.at[pl.ds(80, 64)])
        e0 = [emb_v[0, pl.ds(c * _L, _L)] for c in range(_NCH)]
        e64 = [emb_v[_V - 1, pl.ds(c * _L, _L)] for c in range(_NCH)]

        def lo_fill(x, carry):
            for c in range(_NCH):
                tab_v[x, pl.ds(c * _L, _L)] = e0[c]
            return carry

        def hi_fill(x, carry):
            for c in range(_NCH):
                tab_v[x, pl.ds(c * _L, _L)] = e64[c]
            return carry

        lax.fori_loop(0, 80, lo_fill, 0)
        lax.fori_loop(144, _W, hi_fill, 0)
        pltpu.sync_copy(tab_v, tab_hbm)


_table = functools.partial(
    pl.kernel,
    mesh=plsc.VectorSubcoreMesh(core_axis_name="c", subcore_axis_name="s"),
    out_type=jax.ShapeDtypeStruct((_W, _C), jnp.float32),
    scratch_types=[
        pltpu.VMEM((_V, _C), jnp.float32),
        pltpu.VMEM((_W, _C), jnp.float32),
    ],
)(_sc_body)


def _tc_body(tabt_ref, hcols_ref, out_ref, acc_ref):
    b = pl.program_id(1)

    @pl.when(b == 0)
    def _():
        t = tabt_ref[...]
        hc = hcols_ref[0]
        acc_ref[...] = jnp.concatenate(
            [t + hc[:, i:i + 1] for i in range(_HG)], axis=1)

    out_ref[0] = acc_ref[...]


_fanout = pl.pallas_call(
    _tc_body,
    grid=(_H // _HG, _B),
    in_specs=[
        pl.BlockSpec((_C, _W), lambda k, b: (0, 0)),
        pl.BlockSpec((1, _C, _HG), lambda k, b: (k, 0, 0)),
    ],
    out_specs=pl.BlockSpec((1, _C, _HG * _W), lambda k, b: (b, 0, k)),
    out_shape=jax.ShapeDtypeStruct((_B, _C, _H * _W), jnp.float32),
    scratch_shapes=[pltpu.VMEM((_C, _HG * _W), jnp.float32)],
    compiler_params=pltpu.CompilerParams(
        dimension_semantics=("arbitrary", "arbitrary")),
)


def kernel(feature_map, embedding):
    del feature_map
    tabt = _table(embedding).T
    hcols = tabt.reshape(_C, _H // _HG, _HG).swapaxes(0, 1)
    return jnp.swapaxes(_fanout(tabt, hcols), 1, 2)


# speedup vs baseline: 1.3121x; 1.0473x over previous
import functools

import jax
import jax.numpy as jnp
from jax import lax
from jax.experimental import pallas as pl
from jax.experimental.pallas import tpu as pltpu
from jax.experimental.pallas import tpu_sc as plsc

_MAX = 32
_B, _C, _H, _W = 4, 96, 224, 224
_V = 2 * _MAX + 1
_L = 16
_NCH = _C // _L
_HG = 56


_CPW = 8
_NACT = _C // _CPW
_NK = _H // _HG
_HGP = 64


def _sc_body(emb_hbm, tabt_hbm, hcols_hbm, emb_v, buf_v, hbuf_v):
    cid = lax.axis_index("c")
    sid = lax.axis_index("s")
    wid = sid * 2 + cid
    c0 = wid * _CPW

    def xi_vec(x0):
        return jnp.clip(lax.iota(jnp.int32, _L) + (x0 - _H // 2),
                        -_MAX, _MAX) + _MAX

    @pl.when(wid < _NACT)
    def _():
        pltpu.sync_copy(emb_hbm, emb_v)
        for r in range(_CPW):
            cvec = jnp.zeros((_L,), jnp.int32) + (c0 + r)
            for g in range(_W // _L):
                buf_v[r, pl.ds(g * _L, _L)] = plsc.load_gather(
                    emb_v, [xi_vec(g * _L), cvec])
            for k in range(_NK):
                for g in range(_HGP // _L):
                    hbuf_v[k, r, pl.ds(g * _L, _L)] = plsc.load_gather(
                        emb_v, [xi_vec(k * _HG + g * _L), cvec])
        pltpu.sync_copy(buf_v, tabt_hbm.at[pl.ds(c0, _CPW)])
        pltpu.sync_copy(hbuf_v, hcols_hbm.at[:, pl.ds(c0, _CPW), :])


_table = functools.partial(
    pl.kernel,
    mesh=plsc.VectorSubcoreMesh(core_axis_name="c", subcore_axis_name="s"),
    out_type=(jax.ShapeDtypeStruct((_C, _W), jnp.float32),
              jax.ShapeDtypeStruct((_NK, _C, _HGP), jnp.float32)),
    scratch_types=[
        pltpu.VMEM((_V, _C), jnp.float32),
        pltpu.VMEM((_CPW, _W), jnp.float32),
        pltpu.VMEM((_NK, _CPW, _HGP), jnp.float32),
    ],
    compiler_params=pltpu.CompilerParams(needs_layout_passes=False),
)(_sc_body)


def _tc_body(tabt_ref, hcols_ref, out_ref, acc_ref):
    b = pl.program_id(1)

    @pl.when(b == 0)
    def _():
        t = tabt_ref[...]
        hc = hcols_ref[0]
        acc_ref[...] = jnp.concatenate(
            [t + hc[:, i:i + 1] for i in range(_HG)], axis=1)

    out_ref[0] = acc_ref[...]


_fanout = pl.pallas_call(
    _tc_body,
    grid=(_H // _HG, _B),
    in_specs=[
        pl.BlockSpec((_C, _W), lambda k, b: (0, 0)),
        pl.BlockSpec((1, _C, _HGP), lambda k, b: (k, 0, 0)),
    ],
    out_specs=pl.BlockSpec((1, _C, _HG * _W), lambda k, b: (b, 0, k)),
    out_shape=jax.ShapeDtypeStruct((_B, _C, _H * _W), jnp.float32),
    scratch_shapes=[pltpu.VMEM((_C, _HG * _W), jnp.float32)],
    compiler_params=pltpu.CompilerParams(
        dimension_semantics=("arbitrary", "arbitrary")),
)


def kernel(feature_map, embedding):
    del feature_map
    tabt, hcols = _table(embedding)
    return jnp.swapaxes(_fanout(tabt, hcols), 1, 2)
